# baseline (speedup 1.0000x reference)
import functools

import jax
import jax.numpy as jnp
from jax import lax
from jax.experimental import pallas as pl
from jax.experimental.pallas import tpu as pltpu
from jax.experimental.pallas import tpu_sc as plsc

VOCAB = 1_000_000
HID = 64
BATCH = 16384
LANES = 16

NUM_CORES = 2
NUM_SUBCORES = 16
NW = NUM_CORES * NUM_SUBCORES
BPW = BATCH // NW
NGROUP = BPW // LANES
NSLOTC = 8
NSLOTT = 4

_MESH = plsc.VectorSubcoreMesh(core_axis_name="c", subcore_axis_name="s")


@functools.partial(
    pl.kernel,
    out_type=jax.ShapeDtypeStruct((BATCH,), jnp.float32),
    mesh=_MESH,
    scratch_types=[
        pltpu.VMEM((BPW,), jnp.int32),
        pltpu.VMEM((BPW,), jnp.int32),
        pltpu.VMEM((BPW,), jnp.float32),
        pltpu.VMEM((NSLOTC, HID, 128), jnp.float32),
        pltpu.VMEM((NSLOTT, HID, 128), jnp.float32),
        pltpu.VMEM((BPW,), jnp.float32),
        pltpu.SemaphoreType.DMA((NSLOTC,)),
        pltpu.SemaphoreType.DMA((NSLOTT,)),
    ],
    compiler_params=pltpu.CompilerParams(
        needs_layout_passes=False, use_tc_tiling_on_sc=True),
)
def _skipgram(center_hbm, target_hbm, label_hbm, int_hbm, outt_hbm, y_hbm,
              cidx, tidx, lab, ctile, ttile, yv, csem, tsem):
    wid = lax.axis_index("s") * NUM_CORES + lax.axis_index("c")
    base = wid * BPW

    pltpu.sync_copy(center_hbm.at[pl.ds(base, BPW)], cidx)
    pltpu.sync_copy(target_hbm.at[pl.ds(base, BPW)], tidx)
    pltpu.sync_copy(label_hbm.at[pl.ds(base, BPW)], lab)

    lane = lax.iota(jnp.int32, LANES)
    rowidx = [lane + k * LANES for k in range(HID // LANES)]
    onehot = [jnp.where(lane == r, 1.0, 0.0).astype(jnp.float32)
              for r in range(LANES)]

    def issue(vec, r, tbl, ring, sem, nslot):
        v128 = pl.multiple_of((vec[r] >> 7) << 7, 128)
        for b in range(8):
            pltpu.async_copy(
                tbl.at[pl.ds(8 * b, 8), pl.ds(v128, 128)],
                ring.at[r % nslot, pl.ds(8 * b, 8)], sem.at[r % nslot])

    def waitslot(tbl, ring, sem, r, nslot):
        pltpu.make_async_copy(tbl.at[:, pl.ds(0, 128)], ring.at[r % nslot],
                              sem.at[r % nslot]).wait()

    cvec0 = cidx[pl.ds(0, LANES)]
    tvec0 = tidx[pl.ds(0, LANES)]
    for r in range(NSLOTC):
        issue(cvec0, r, int_hbm, ctile, csem, NSLOTC)
    for r in range(NSLOTT):
        issue(tvec0, r, outt_hbm, ttile, tsem, NSLOTT)

    def group_body(g, _):
        gb = g * LANES
        sl = pl.ds(gb, LANES)
        cvec = cidx[sl]
        tvec = tidx[sl]
        labv = lab[sl]
        nb = jnp.minimum(gb + LANES, BPW - LANES)
        ncvec = cidx[pl.ds(nb, LANES)]
        ntvec = tidx[pl.ds(nb, LANES)]
        res = jnp.zeros((LANES,), jnp.float32)
        for r in range(LANES):
            waitslot(int_hbm, ctile, csem, r, NSLOTC)
            waitslot(outt_hbm, ttile, tsem, r, NSLOTT)
            ccol = jnp.broadcast_to(cvec[r] & 127, (LANES,))
            tcol = jnp.broadcast_to(tvec[r] & 127, (LANES,))
            cslot = ctile.at[r % NSLOTC]
            tslot = ttile.at[r % NSLOTT]
            acc = None
            for k in range(HID // LANES):
                cg = plsc.load_gather(cslot, [rowidx[k], ccol])
                tg = plsc.load_gather(tslot, [rowidx[k], tcol])
                p = cg * tg
                acc = p if acc is None else acc + p
            res = res + jnp.sum(acc) * onehot[r]
            for j, tbl, ring, sem, ns, nvec, cur in (
                    (r + NSLOTC, int_hbm, ctile, csem, NSLOTC, ncvec, cvec),
                    (r + NSLOTT, outt_hbm, ttile, tsem, NSLOTT, ntvec, tvec)):
                if j < LANES:
                    issue(cur, j, tbl, ring, sem, ns)
                else:

                    @pl.when(g < NGROUP - 1)
                    def _(j=j, tbl=tbl, ring=ring, sem=sem, ns=ns, nvec=nvec):
                        issue(nvec, j - LANES, tbl, ring, sem, ns)

        yv[sl] = res * labv
        return 0

    lax.fori_loop(0, NGROUP, group_body, 0)

    pltpu.sync_copy(yv, y_hbm.at[pl.ds(base, BPW)])


def kernel(center, target, label, in_emb, out_emb):
    center = center.astype(jnp.int32)
    target = target.astype(jnp.int32)
    return _skipgram(center, target, label, in_emb.T, out_emb.T)


# speedup vs baseline: 1.0082x; 1.0082x over previous
import functools

import jax
import jax.numpy as jnp
from jax import lax
from jax.experimental import pallas as pl
from jax.experimental.pallas import tpu as pltpu
from jax.experimental.pallas import tpu_sc as plsc

VOCAB = 1_000_000
HID = 64
BATCH = 16384
LANES = 16

NUM_CORES = 2
NUM_SUBCORES = 16
NW = NUM_CORES * NUM_SUBCORES
BPW = BATCH // NW
NGROUP = BPW // LANES
NSLOTC = 8
NSLOTT = 4

_MESH = plsc.VectorSubcoreMesh(core_axis_name="c", subcore_axis_name="s")


@functools.partial(
    pl.kernel,
    out_type=jax.ShapeDtypeStruct((BATCH,), jnp.float32),
    mesh=_MESH,
    scratch_types=[
        pltpu.VMEM((BPW,), jnp.int32),
        pltpu.VMEM((BPW,), jnp.int32),
        pltpu.VMEM((BPW,), jnp.float32),
        pltpu.VMEM((NSLOTC, HID, 128), jnp.float32),
        pltpu.VMEM((NSLOTT, HID, 128), jnp.float32),
        pltpu.VMEM((BPW,), jnp.float32),
        pltpu.SemaphoreType.DMA((NSLOTC,)),
        pltpu.SemaphoreType.DMA((NSLOTT,)),
    ],
    compiler_params=pltpu.CompilerParams(
        needs_layout_passes=False, use_tc_tiling_on_sc=True),
)
def _skipgram(center_hbm, target_hbm, label_hbm, int_hbm, outt_hbm, y_hbm,
              cidx, tidx, lab, ctile, ttile, yv, csem, tsem):
    wid = lax.axis_index("s") * NUM_CORES + lax.axis_index("c")
    base = wid * BPW

    pltpu.sync_copy(center_hbm.at[pl.ds(base, BPW)], cidx)
    pltpu.sync_copy(target_hbm.at[pl.ds(base, BPW)], tidx)
    pltpu.sync_copy(label_hbm.at[pl.ds(base, BPW)], lab)

    lane = lax.iota(jnp.int32, LANES)
    rowidx = [lane + k * LANES for k in range(HID // LANES)]
    onehot = [jnp.where(lane == r, 1.0, 0.0).astype(jnp.float32)
              for r in range(LANES)]

    def issue(vec, r, tbl, ring, sem, nslot):
        v128 = pl.multiple_of((vec[r] >> 7) << 7, 128)
        pltpu.async_copy(tbl.at[:, pl.ds(v128, 128)], ring.at[r % nslot],
                         sem.at[r % nslot])

    def waitslot(tbl, ring, sem, r, nslot):
        pltpu.make_async_copy(tbl.at[:, pl.ds(0, 128)], ring.at[r % nslot],
                              sem.at[r % nslot]).wait()

    cvec0 = cidx[pl.ds(0, LANES)]
    tvec0 = tidx[pl.ds(0, LANES)]
    for r in range(NSLOTC):
        issue(cvec0, r, int_hbm, ctile, csem, NSLOTC)
    for r in range(NSLOTT):
        issue(tvec0, r, outt_hbm, ttile, tsem, NSLOTT)

    def group_body(g, _):
        gb = g * LANES
        sl = pl.ds(gb, LANES)
        cvec = cidx[sl]
        tvec = tidx[sl]
        labv = lab[sl]
        nb = jnp.minimum(gb + LANES, BPW - LANES)
        ncvec = cidx[pl.ds(nb, LANES)]
        ntvec = tidx[pl.ds(nb, LANES)]
        res = jnp.zeros((LANES,), jnp.float32)
        for r in range(LANES):
            waitslot(int_hbm, ctile, csem, r, NSLOTC)
            waitslot(outt_hbm, ttile, tsem, r, NSLOTT)
            ccol = jnp.broadcast_to(cvec[r] & 127, (LANES,))
            tcol = jnp.broadcast_to(tvec[r] & 127, (LANES,))
            cslot = ctile.at[r % NSLOTC]
            tslot = ttile.at[r % NSLOTT]
            acc = None
            for k in range(HID // LANES):
                cg = plsc.load_gather(cslot, [rowidx[k], ccol])
                tg = plsc.load_gather(tslot, [rowidx[k], tcol])
                p = cg * tg
                acc = p if acc is None else acc + p
            res = res + jnp.sum(acc) * onehot[r]
            for j, tbl, ring, sem, ns, nvec, cur in (
                    (r + NSLOTC, int_hbm, ctile, csem, NSLOTC, ncvec, cvec),
                    (r + NSLOTT, outt_hbm, ttile, tsem, NSLOTT, ntvec, tvec)):
                if j < LANES:
                    issue(cur, j, tbl, ring, sem, ns)
                else:

                    @pl.when(g < NGROUP - 1)
                    def _(j=j, tbl=tbl, ring=ring, sem=sem, ns=ns, nvec=nvec):
                        issue(nvec, j - LANES, tbl, ring, sem, ns)

        yv[sl] = res * labv
        return 0

    lax.fori_loop(0, NGROUP, group_body, 0)

    pltpu.sync_copy(yv, y_hbm.at[pl.ds(base, BPW)])


def kernel(center, target, label, in_emb, out_emb):
    center = center.astype(jnp.int32)
    target = target.astype(jnp.int32)
    return _skipgram(center, target, label, in_emb.T, out_emb.T)
